# Initial kernel scaffold; baseline (speedup 1.0000x reference)
import jax
import jax.numpy as jnp
from jax.experimental import pallas as pl
from jax.experimental.pallas import tpu as pltpu

H = 32
W_ = 32
P = H * W_
CIN = 64
COUT = 64
KK = 9


def _body(xt_ref, wt_ref, idx_ref, out_ref):
    xt = xt_ref[0]
    acc = jnp.zeros((P, COUT), jnp.float32)
    for k in range(KK):
        y = jnp.dot(xt, wt_ref[k], preferred_element_type=jnp.float32)
        idx = idx_ref[k, 0, :]
        acc = acc + jnp.take(y, idx, axis=0)
    out_ref[0] = acc.T


def kernel(x, W, sample_idx):
    B = x.shape[0]
    xt = jnp.transpose(x.reshape(B, CIN, P), (0, 2, 1))
    wt = jnp.transpose(W.reshape(COUT, CIN, KK), (2, 1, 0))
    idx = jnp.transpose(sample_idx.reshape(P, KK), (1, 0)).reshape(KK, 1, P)

    out = pl.pallas_call(
        _body,
        grid=(B,),
        in_specs=[
            pl.BlockSpec((1, P, CIN), lambda b: (b, 0, 0)),
            pl.BlockSpec((KK, CIN, COUT), lambda b: (0, 0, 0)),
            pl.BlockSpec((KK, 1, P), lambda b: (0, 0, 0)),
        ],
        out_specs=pl.BlockSpec((1, COUT, P), lambda b: (b, 0, 0)),
        out_shape=jax.ShapeDtypeStruct((B, COUT, P), jnp.float32),
    )(xt, wt, idx)
    return out.reshape(B, COUT, H, W_)


# speedup vs baseline: 1.4191x; 1.4191x over previous
import jax
import jax.numpy as jnp
from jax.experimental import pallas as pl
from jax.experimental.pallas import tpu as pltpu

H = 32
W_ = 32
P = H * W_
CIN = 64
COUT = 64
KK = 9


def _body(x_ref, w_ref, idx_ref, out_ref, s_ref):
    @pl.when(pl.program_id(0) == 0)
    def _build_onehot():
        iq = jax.lax.broadcasted_iota(jnp.int32, (P, P), 0)
        for k in range(KK):
            pk = idx_ref[k]
            s_ref[k] = jnp.where(iq == pk, 1.0, 0.0).astype(jnp.bfloat16)

    x = x_ref[0].astype(jnp.bfloat16)
    z = jnp.dot(w_ref[...], x, preferred_element_type=jnp.float32)
    zb = z.astype(jnp.bfloat16)
    acc = jnp.zeros((COUT, P), jnp.float32)
    for k in range(KK):
        acc = acc + jnp.dot(
            zb[k * COUT:(k + 1) * COUT], s_ref[k],
            preferred_element_type=jnp.float32)
    out_ref[0] = acc


def kernel(x, W, sample_idx):
    B = x.shape[0]
    wstack = jnp.transpose(W.reshape(COUT, CIN, KK), (2, 0, 1)).reshape(
        KK * COUT, CIN).astype(jnp.bfloat16)
    idx = jnp.transpose(sample_idx.reshape(P, KK), (1, 0)).reshape(KK, 1, P)

    out = pl.pallas_call(
        _body,
        grid=(B,),
        in_specs=[
            pl.BlockSpec((1, CIN, P), lambda b: (b, 0, 0)),
            pl.BlockSpec((KK * COUT, CIN), lambda b: (0, 0)),
            pl.BlockSpec((KK, 1, P), lambda b: (0, 0, 0)),
        ],
        out_specs=pl.BlockSpec((1, COUT, P), lambda b: (b, 0, 0)),
        out_shape=jax.ShapeDtypeStruct((B, COUT, P), jnp.float32),
        scratch_shapes=[pltpu.VMEM((KK, P, P), jnp.bfloat16)],
    )(x.reshape(B, CIN, P), wstack, idx)
    return out.reshape(B, COUT, H, W_)
